# baseline (speedup 1.0000x reference)
import functools
import math

import jax
import jax.numpy as jnp
from jax import lax
from jax.experimental import pallas as pl
from jax.experimental.pallas import tpu as pltpu
from jax.experimental.pallas import tpu_sc as plsc

_NUM_CORES = 2
_NUM_SUBCORES = 16
_NW = _NUM_CORES * _NUM_SUBCORES
_LANES = 16


def _positional_encoding(max_len, d_model):
    pos = jnp.arange(max_len, dtype=jnp.float32)[:, None]
    index = jnp.arange(d_model, dtype=jnp.float32)[None, :]
    pe = pos / jnp.power(10000.0, (index - index % 2) / float(d_model))
    pe_s = jnp.sin(pe[:, 0::2])[..., None]
    pe_c = jnp.cos(pe[:, 1::2])[..., None]
    return jnp.concatenate([pe_s, pe_c], axis=-1).reshape(pe.shape[0], -1)


@functools.partial(jax.jit, static_argnames=("seq_len",))
def _lookup(idx2d, table, pe, seq_len):
    n_sub, ksub = idx2d.shape
    v, d = table.shape
    n = n_sub * ksub
    per_w = n // _NW
    ch = seq_len
    n_ch = per_w // ch
    sub_per_ch = ch // ksub
    scale = float(math.sqrt(d))
    mesh = plsc.VectorSubcoreMesh(core_axis_name="c", subcore_axis_name="s")

    @functools.partial(
        pl.kernel,
        out_type=jax.ShapeDtypeStruct((n, d), jnp.float32),
        mesh=mesh,
        compiler_params=pltpu.CompilerParams(use_tc_tiling_on_sc=False),
        scratch_types=[
            pltpu.VMEM((per_w // ksub, ksub), jnp.int32),
            pltpu.VMEM((ch, d), jnp.float32),
            pltpu.VMEM((ch, d), jnp.float32),
            pltpu.SemaphoreType.DMA,
        ],
    )
    def k(table_hbm, idx_hbm, pe_hbm, out_hbm, idx_v, rows_v, pe_v, sem):
        wid = lax.axis_index("s") * _NUM_CORES + lax.axis_index("c")
        pltpu.sync_copy(pe_hbm, pe_v)
        pltpu.sync_copy(
            idx_hbm.at[pl.ds(wid * (per_w // ksub), per_w // ksub)], idx_v
        )

        def chunk_body(c, carry):
            row0 = wid * per_w + c * ch
            cps = [
                pltpu.async_copy(
                    table_hbm.at[idx_v.at[c * sub_per_ch + j]],
                    rows_v.at[pl.ds(j * ksub, ksub)],
                    sem,
                )
                for j in range(sub_per_ch)
            ]
            for cp in cps:
                cp.wait()

            def row_body(r, carry2):
                for t in range(d // _LANES):
                    sl = pl.ds(t * _LANES, _LANES)
                    rows_v[r, sl] = rows_v[r, sl] * scale + pe_v[r, sl]
                return carry2

            lax.fori_loop(0, ch, row_body, 0, unroll=2)
            pltpu.sync_copy(rows_v, out_hbm.at[pl.ds(row0, ch)])
            return carry

        lax.fori_loop(0, n_ch, chunk_body, 0)

    return k(table, idx2d, pe)


def kernel(sequences, table):
    b, s = sequences.shape
    v, d = table.shape
    n = b * s
    ksub = s // 2
    idx2d = sequences.astype(jnp.int32).reshape(n // ksub, ksub)
    pe = _positional_encoding(s, d)
    out = _lookup(idx2d, table, pe, s)
    return out.reshape(b, s, d)


# speedup vs baseline: 1.1786x; 1.1786x over previous
import functools
import math

import jax
import jax.numpy as jnp
from jax import lax
from jax.experimental import pallas as pl
from jax.experimental.pallas import tpu as pltpu
from jax.experimental.pallas import tpu_sc as plsc

_NUM_CORES = 2
_NUM_SUBCORES = 16
_NW = _NUM_CORES * _NUM_SUBCORES
_LANES = 16
_NBUF = 4
_SEQ_PER_CHUNK = 2


def _positional_encoding(max_len, d_model):
    pos = jnp.arange(max_len, dtype=jnp.float32)[:, None]
    index = jnp.arange(d_model, dtype=jnp.float32)[None, :]
    pe = pos / jnp.power(10000.0, (index - index % 2) / float(d_model))
    pe_s = jnp.sin(pe[:, 0::2])[..., None]
    pe_c = jnp.cos(pe[:, 1::2])[..., None]
    return jnp.concatenate([pe_s, pe_c], axis=-1).reshape(pe.shape[0], -1)


@functools.partial(jax.jit, static_argnames=("seq_len",))
def _lookup(idx2d, table, pe, seq_len):
    n_sub, ksub = idx2d.shape
    v, d = table.shape
    n = n_sub * ksub
    per_w = n // _NW
    ch = _SEQ_PER_CHUNK * seq_len
    n_ch = per_w // ch
    sub_per_ch = ch // ksub
    idx_rows_w = per_w // ksub
    scale = float(math.sqrt(d))
    mesh = plsc.VectorSubcoreMesh(core_axis_name="c", subcore_axis_name="s")

    @functools.partial(
        pl.kernel,
        out_type=jax.ShapeDtypeStruct((n, d), jnp.float32),
        mesh=mesh,
        compiler_params=pltpu.CompilerParams(use_tc_tiling_on_sc=False),
        scratch_types=[
            pltpu.VMEM((idx_rows_w, ksub), jnp.int32),
            pltpu.VMEM((_NBUF, ch, d), jnp.float32),
            pltpu.VMEM((seq_len, d), jnp.float32),
            [pltpu.SemaphoreType.DMA] * _NBUF,
            [pltpu.SemaphoreType.DMA] * _NBUF,
        ],
    )
    def k(table_hbm, idx_hbm, pe_hbm, out_hbm, idx_v, rows_v, pe_v, gsems, ssems):
        wid = lax.axis_index("s") * _NUM_CORES + lax.axis_index("c")
        pltpu.sync_copy(pe_hbm, pe_v)
        pltpu.sync_copy(idx_hbm.at[pl.ds(wid * idx_rows_w, idx_rows_w)], idx_v)

        def start_gather(c, b):
            for j in range(sub_per_ch):
                pltpu.async_copy(
                    table_hbm.at[idx_v.at[c * sub_per_ch + j]],
                    rows_v.at[b].at[pl.ds(j * ksub, ksub)],
                    gsems[b],
                )

        def wait_gather(b):
            pltpu.make_async_copy(
                out_hbm.at[pl.ds(0, ch)], rows_v.at[b], gsems[b]
            ).wait()

        def start_scatter(c, b):
            row0 = wid * per_w + c * ch
            pltpu.async_copy(rows_v.at[b], out_hbm.at[pl.ds(row0, ch)], ssems[b])

        def wait_scatter(b):
            pltpu.make_async_copy(
                rows_v.at[b], out_hbm.at[pl.ds(0, ch)], ssems[b]
            ).wait()

        def compute(b):
            buf = rows_v.at[b]

            @plsc.parallel_loop(0, seq_len, unroll=2)
            def _(p):
                for t in range(d // _LANES):
                    sl = pl.ds(t * _LANES, _LANES)
                    pe_val = pe_v[p, sl]
                    buf[p, sl] = buf[p, sl] * scale + pe_val
                    q = p + seq_len
                    buf[q, sl] = buf[q, sl] * scale + pe_val

        start_gather(0, 0)
        start_gather(1, 1)

        def outer(o, carry):
            for b in range(_NBUF):
                c = o * _NBUF + b
                bn = (b + 2) % _NBUF

                @pl.when(c + 2 < n_ch)
                def _():
                    @pl.when(c >= 2)
                    def _():
                        wait_scatter(bn)

                    start_gather(c + 2, bn)

                wait_gather(b)
                compute(b)
                start_scatter(c, b)
            return carry

        lax.fori_loop(0, n_ch // _NBUF, outer, 0)
        wait_scatter((n_ch - 2) % _NBUF)
        wait_scatter((n_ch - 1) % _NBUF)

    return k(table, idx2d, pe)


def kernel(sequences, table):
    b, s = sequences.shape
    v, d = table.shape
    n = b * s
    ksub = s // 2
    idx2d = sequences.astype(jnp.int32).reshape(n // ksub, ksub)
    pe = _positional_encoding(s, d)
    out = _lookup(idx2d, table, pe, s)
    return out.reshape(b, s, d)
